# baseline (speedup 1.0000x reference)
import functools

import jax
import jax.numpy as jnp
from jax import lax
from jax.experimental import pallas as pl
from jax.experimental.pallas import tpu as pltpu
from jax.experimental.pallas import tpu_sc as plsc

_KE_KCAL = 332.06371
_R_CUT = 5.0
_R_ON = _R_CUT / 4.0
_R_OFF = 3.0 * _R_CUT / 4.0
_N = 2048
_TILE = 256
_NW = 32
_CHUNK = _N // _NW


def _qz_kernel(zt_hbm, z_hbm, out_hbm, idx_v, rows_v, sem):
    wid = lax.axis_index("s") * 2 + lax.axis_index("c")
    base = wid * _CHUNK
    pltpu.sync_copy(z_hbm.at[pl.ds(base, _CHUNK)], idx_v)
    pltpu.async_copy(zt_hbm.at[idx_v], rows_v, sem).wait()
    pltpu.sync_copy(rows_v, out_hbm.at[pl.ds(base, _CHUNK)])


def _energy_kernel(qz_ref, f_ref, w_ref, tc_ref, xyz_ref, xyzt_ref,
                   energy_ref, q_ref):
    p = pl.program_id(0)

    @pl.when(p == 0)
    def _():
        w_f = jnp.dot(f_ref[...], w_ref[...], preferred_element_type=jnp.float32)
        pred = w_f + qz_ref[...]
        corr = (tc_ref[...] - jnp.sum(pred)) * (1.0 / _N)
        q_ref[...] = pred + corr
        energy_ref[...] = jnp.zeros((1, 1), jnp.float32)

    nb = _N // _TILE
    i_a = p
    i_b = nb - 1 - p
    n_a = nb - i_a
    iota0 = lax.broadcasted_iota(jnp.int32, (_TILE, _TILE), 0)
    iota1 = lax.broadcasted_iota(jnp.int32, (_TILE, _TILE), 1)
    inv_w = 1.0 / (_R_OFF - _R_ON)

    def block(t, e):
        is_a = t < n_a
        i0 = jnp.where(is_a, i_a, i_b) * _TILE
        j0 = jnp.where(is_a, i_a + t, t - 1) * _TILE
        tile = xyz_ref[pl.ds(i0, _TILE), :]
        rows = xyzt_ref[:, pl.ds(j0, _TILE)]
        dx = tile[:, 0:1] - rows[0:1, :]
        dy = tile[:, 1:2] - rows[1:2, :]
        dz = tile[:, 2:3] - rows[2:3, :]
        d2 = dx * dx + dy * dy + dz * dz
        mask = ((j0 - i0 + iota1 - iota0) > 0) & (d2 > 0)

        sd2 = jnp.where(mask, d2, 1.0)
        rinv = lax.rsqrt(sd2)
        r = sd2 * rinv
        a = jnp.clip((r - _R_ON) * inv_w, 1e-4, 1.0 - 1e-4)
        expo = (2.0 * a - 1.0) / (a - a * a)
        fs = 1.0 / (1.0 + jnp.exp(expo))
        isq = lax.rsqrt(sd2 + 1.0)
        g = fs * (isq - rinv) + rinv
        g = jnp.where(mask, g, 0.0)
        qj = q_ref[pl.ds(j0, _TILE), :]
        ev = jnp.dot(g, qj, preferred_element_type=jnp.float32)
        qi = q_ref[pl.ds(i0, _TILE), :]
        return e + jnp.sum(qi * ev, keepdims=True)

    e = lax.fori_loop(0, nb + 1, block, jnp.zeros((1, 1), jnp.float32),
                      unroll=3)
    energy_ref[...] += _KE_KCAL * e


@jax.jit
def kernel(f, z, xyz, total_charge, W, z_table):
    zt_pad = jnp.zeros((128, 128), jnp.float32).at[: z_table.shape[0], 0:1].set(
        z_table)
    qz_full = pl.kernel(
        _qz_kernel,
        out_type=jax.ShapeDtypeStruct((_N, 128), jnp.float32),
        mesh=plsc.VectorSubcoreMesh(core_axis_name="c", subcore_axis_name="s"),
        scratch_types=[
            pltpu.VMEM((_CHUNK,), jnp.int32),
            pltpu.VMEM((_CHUNK, 128), jnp.float32),
            pltpu.SemaphoreType.DMA,
        ],
    )(zt_pad, z.astype(jnp.int32))
    qz = qz_full[:, 0:1]

    tc = total_charge.reshape(1, 1)
    xyzt = xyz.T
    energy, q = pl.pallas_call(
        _energy_kernel,
        grid=(_N // _TILE // 2,),
        out_shape=(
            jax.ShapeDtypeStruct((1, 1), jnp.float32),
            jax.ShapeDtypeStruct((_N, 1), jnp.float32),
        ),
    )(qz, f, W, tc, xyz, xyzt)

    return (energy[0, 0], q)


# speedup vs baseline: 1.7818x; 1.7818x over previous
import functools

import jax
import jax.numpy as jnp
from jax import lax
from jax.experimental import pallas as pl
from jax.experimental.pallas import tpu as pltpu
from jax.experimental.pallas import tpu_sc as plsc

_KE_KCAL = 332.06371
_R_CUT = 5.0
_R_ON = _R_CUT / 4.0
_R_OFF = 3.0 * _R_CUT / 4.0
_N = 2048
_TILE = 512
_NW = 32
_CHUNK = _N // _NW


def _qz_kernel(zt_hbm, z_hbm, out_hbm, idx_v, rows_v, sem):
    wid = lax.axis_index("s") * 2 + lax.axis_index("c")
    base = wid * _CHUNK
    pltpu.sync_copy(z_hbm.at[pl.ds(base, _CHUNK)], idx_v)
    pltpu.async_copy(zt_hbm.at[idx_v], rows_v, sem).wait()
    pltpu.sync_copy(rows_v, out_hbm.at[pl.ds(base, _CHUNK)])


def _energy_kernel(f_ref, z_ref, w_ref, zt_ref, tc_ref, xyz_ref, xyzt_ref,
                   energy_ref, q_ref):
    p = pl.program_id(0)

    @pl.when(p == 0)
    def _():
        w_f = jnp.dot(f_ref[...], w_ref[...], preferred_element_type=jnp.float32)
        lane = lax.broadcasted_iota(jnp.int32, (_N, 128), 1)
        onehot = (z_ref[...] == lane).astype(jnp.float32)
        q_z = jnp.dot(onehot, zt_ref[...], preferred_element_type=jnp.float32)
        pred = w_f + q_z
        corr = (tc_ref[...] - jnp.sum(pred)) * (1.0 / _N)
        q_ref[...] = pred + corr
        energy_ref[...] = jnp.zeros((1, 1), jnp.float32)

    nb = _N // _TILE
    i_a = p
    i_b = nb - 1 - p
    n_a = nb - i_a
    iota0 = lax.broadcasted_iota(jnp.int32, (_TILE, _TILE), 0)
    iota1 = lax.broadcasted_iota(jnp.int32, (_TILE, _TILE), 1)
    inv_w = 1.0 / (_R_OFF - _R_ON)

    def block(t, e):
        is_a = t < n_a
        i0 = jnp.where(is_a, i_a, i_b) * _TILE
        j0 = jnp.where(is_a, i_a + t, t - 1) * _TILE
        tile = xyz_ref[pl.ds(i0, _TILE), :]
        rows = xyzt_ref[:, pl.ds(j0, _TILE)]
        dx = tile[:, 0:1] - rows[0:1, :]
        dy = tile[:, 1:2] - rows[1:2, :]
        dz = tile[:, 2:3] - rows[2:3, :]
        d2 = dx * dx + dy * dy + dz * dz
        mask = ((j0 - i0 + iota1 - iota0) > 0) & (d2 > 0)

        sd2 = jnp.where(mask, d2, 1.0)
        rinv = lax.rsqrt(sd2)
        r = sd2 * rinv
        a = jnp.clip((r - _R_ON) * inv_w, 1e-4, 1.0 - 1e-4)
        expo = (2.0 * a - 1.0) / (a - a * a)
        fs = 1.0 / (1.0 + jnp.exp(expo))
        isq = lax.rsqrt(sd2 + 1.0)
        g = fs * (isq - rinv) + rinv
        g = jnp.where(mask, g, 0.0)
        qj = q_ref[pl.ds(j0, _TILE), :]
        ev = jnp.dot(g, qj, preferred_element_type=jnp.float32)
        qi = q_ref[pl.ds(i0, _TILE), :]
        return e + jnp.sum(qi * ev, keepdims=True)

    e = lax.fori_loop(0, nb + 1, block, jnp.zeros((1, 1), jnp.float32),
                      unroll=nb + 1)
    energy_ref[...] += _KE_KCAL * e


@jax.jit
def kernel(f, z, xyz, total_charge, W, z_table):
    z2d = z.astype(jnp.int32).reshape(_N, 1)
    zt_pad = jnp.zeros((128, 1), jnp.float32).at[: z_table.shape[0]].set(z_table)
    tc = total_charge.reshape(1, 1)
    xyzt = xyz.T
    energy, q = pl.pallas_call(
        _energy_kernel,
        grid=(_N // _TILE // 2,),
        out_shape=(
            jax.ShapeDtypeStruct((1, 1), jnp.float32),
            jax.ShapeDtypeStruct((_N, 1), jnp.float32),
        ),
    )(f, z2d, W, zt_pad, tc, xyz, xyzt)

    return (energy[0, 0], q)


# speedup vs baseline: 2.0163x; 1.1316x over previous
import functools

import jax
import jax.numpy as jnp
from jax import lax
from jax.experimental import pallas as pl
from jax.experimental.pallas import tpu as pltpu
from jax.experimental.pallas import tpu_sc as plsc

_KE_KCAL = 332.06371
_R_CUT = 5.0
_R_ON = _R_CUT / 4.0
_R_OFF = 3.0 * _R_CUT / 4.0
_N = 2048
_TILE = 512
_NW = 32
_CHUNK = _N // _NW


def _qz_kernel(zt_hbm, z_hbm, out_hbm, idx_v, rows_v, sem):
    wid = lax.axis_index("s") * 2 + lax.axis_index("c")
    base = wid * _CHUNK
    pltpu.sync_copy(z_hbm.at[pl.ds(base, _CHUNK)], idx_v)
    pltpu.async_copy(zt_hbm.at[idx_v], rows_v, sem).wait()
    pltpu.sync_copy(rows_v, out_hbm.at[pl.ds(base, _CHUNK)])


def _energy_kernel(f_ref, z_ref, w_ref, zt_ref, tc_ref, xyz_ref, xyzt_ref,
                   energy_ref, q_ref):
    p = pl.program_id(0)

    @pl.when(p == 0)
    def _():
        w_f = jnp.dot(f_ref[...], w_ref[...], preferred_element_type=jnp.float32)
        lane = lax.broadcasted_iota(jnp.int32, (_N, 128), 1)
        onehot = (z_ref[...] == lane).astype(jnp.float32)
        q_z = jnp.dot(onehot, zt_ref[...], preferred_element_type=jnp.float32)
        pred = w_f + q_z
        corr = (tc_ref[...] - jnp.sum(pred)) * (1.0 / _N)
        q_ref[...] = pred + corr
        energy_ref[...] = jnp.zeros((1, 1), jnp.float32)

    nb = _N // _TILE
    i_a = p
    i_b = nb - 1 - p
    n_a = nb - i_a
    iota0 = lax.broadcasted_iota(jnp.int32, (_TILE, _TILE), 0)
    iota1 = lax.broadcasted_iota(jnp.int32, (_TILE, _TILE), 1)
    inv_w = 1.0 / (_R_OFF - _R_ON)

    def block(t, e):
        is_a = t < n_a
        i0 = jnp.where(is_a, i_a, i_b) * _TILE
        j0 = jnp.where(is_a, i_a + t, t - 1) * _TILE
        tile = xyz_ref[pl.ds(i0, _TILE), :]
        rows = xyzt_ref[:, pl.ds(j0, _TILE)]
        ni = jnp.sum(tile * tile, axis=1, keepdims=True)
        nj = jnp.sum(rows * rows, axis=0, keepdims=True)
        cross2 = jnp.dot(tile * (-2.0), rows,
                         preferred_element_type=jnp.float32)
        d2 = (cross2 + ni) + nj
        mask = ((j0 - i0 + iota1 - iota0) > 0) & (d2 > 0)

        sd2 = d2
        rinv = lax.rsqrt(sd2)
        r = sd2 * rinv
        a = jnp.clip((r - _R_ON) * inv_w, 1e-4, 1.0 - 1e-4)
        expo = (2.0 * a - 1.0) / (a - a * a)
        fs = 1.0 / (1.0 + jnp.exp(expo))
        isq = lax.rsqrt(sd2 + 1.0)
        g = fs * (isq - rinv) + rinv
        g = jnp.where(mask, g, 0.0)
        qj = q_ref[pl.ds(j0, _TILE), :]
        ev = jnp.dot(g, qj, preferred_element_type=jnp.float32)
        qi = q_ref[pl.ds(i0, _TILE), :]
        return e + jnp.sum(qi * ev, keepdims=True)

    e = lax.fori_loop(0, nb + 1, block, jnp.zeros((1, 1), jnp.float32),
                      unroll=nb + 1)
    energy_ref[...] += _KE_KCAL * e


@jax.jit
def kernel(f, z, xyz, total_charge, W, z_table):
    z2d = z.astype(jnp.int32).reshape(_N, 1)
    zt_pad = jnp.zeros((128, 1), jnp.float32).at[: z_table.shape[0]].set(z_table)
    tc = total_charge.reshape(1, 1)
    xyzt = xyz.T
    energy, q = pl.pallas_call(
        _energy_kernel,
        grid=(_N // _TILE // 2,),
        out_shape=(
            jax.ShapeDtypeStruct((1, 1), jnp.float32),
            jax.ShapeDtypeStruct((_N, 1), jnp.float32),
        ),
    )(f, z2d, W, zt_pad, tc, xyz, xyzt)

    return (energy[0, 0], q)
